# Initial kernel scaffold; baseline (speedup 1.0000x reference)
import jax
import jax.numpy as jnp
from jax.experimental import pallas as pl


def kernel(features_tensor, outputs, classes_prototypes):
    raise NotImplementedError("write your pallas kernel here")


# speedup vs baseline: 24.7121x; 24.7121x over previous
import functools
import math

import jax
import jax.numpy as jnp
import numpy as np
from jax import lax
from jax.experimental import pallas as pl
from jax.experimental.pallas import tpu as pltpu
from jax.experimental.pallas import tpu_sc as plsc

H, W = 135, 240
HW = H * W
R = 3600
D = 512
NUM_CLASSES = 19
NEIB = 9
CPAD = 128
FULL_ROW, FULL_COL = 1080, 1920
FULL = FULL_ROW * FULL_COL
BLK = 400
GRID = R // BLK
NC, NS = 2, 16
NW = NC * NS
RPAD = 3840
B_PER_W = RPAD // NW
REGION = FULL // NW
NEG = -1e30
BIG = 3e38

_SAMPLE_INDEX = np.asarray(
    jnp.sort(jax.random.permutation(jax.random.key(42), HW)[:R]),
    dtype=np.int32)
_SAMPLE_INDEX_PAD = np.zeros((RPAD,), np.int32)
_SAMPLE_INDEX_PAD[:R] = _SAMPLE_INDEX

_MESH = dict(core_axis_name="c", subcore_axis_name="s")


def _sc_gather(feats_hw, logits_hw, idx_pad):
    @functools.partial(
        pl.kernel,
        out_type=[jax.ShapeDtypeStruct((RPAD, D), jnp.float32),
                  jax.ShapeDtypeStruct((RPAD, CPAD), jnp.float32)],
        mesh=plsc.VectorSubcoreMesh(**_MESH),
        scratch_types=[pltpu.VMEM((B_PER_W,), jnp.int32),
                       pltpu.VMEM((B_PER_W, D), jnp.float32),
                       pltpu.VMEM((B_PER_W, CPAD), jnp.float32),
                       pltpu.SemaphoreType.DMA,
                       pltpu.SemaphoreType.DMA],
        compiler_params=pltpu.CompilerParams(needs_layout_passes=False),
    )
    def gk(feats_hbm, logits_hbm, idx_hbm, out_f, out_l, idx_v, rows_f,
           rows_l, sem_f, sem_l):
        wid = lax.axis_index("s") * NC + lax.axis_index("c")
        base = pl.multiple_of(wid * B_PER_W, B_PER_W)
        pltpu.sync_copy(idx_hbm.at[pl.ds(base, B_PER_W)], idx_v)
        cp_f = pltpu.async_copy(feats_hbm.at[idx_v], rows_f, sem_f)
        cp_l = pltpu.async_copy(logits_hbm.at[idx_v], rows_l, sem_l)
        cp_f.wait()
        cp_l.wait()
        pltpu.sync_copy(rows_f, out_f.at[pl.ds(base, B_PER_W)])
        pltpu.sync_copy(rows_l, out_l.at[pl.ds(base, B_PER_W)])

    return gk(feats_hw, logits_hw, idx_pad)


def _first_argmax(x, cols):
    rowmax = jnp.max(x, axis=1, keepdims=True)
    return jnp.min(jnp.where(x == rowmax, cols, CPAD), axis=1, keepdims=True)


def _tc_body(feats_ref, logits_ref, protos_ref, pur_ref, ent_ref):
    i = pl.program_id(0)
    K = feats_ref[:R, :]
    Q = feats_ref[pl.ds(i * BLK, BLK), :]
    ones_row = jnp.ones((1, D), jnp.float32)
    dn = (((1,), (1,)), ((), ()))
    hi = lax.Precision.HIGHEST

    mm = lax.dot_general(Q, K, dn)
    qn = jnp.sum(Q * Q, axis=1, keepdims=True)
    kn = lax.dot_general(ones_row, K * K, dn, precision=hi)
    S = (-2.0 * mm + qn) + kn
    S = jnp.maximum(S, 1e-12)

    s_work = S
    m = jnp.min(s_work, axis=1, keepdims=True)
    for _ in range(NEIB - 1):
        s_work = jnp.where(s_work <= m, BIG, s_work)
        m = jnp.min(s_work, axis=1, keepdims=True)
    nmask = (S <= m).astype(jnp.float32)

    L = logits_ref[:R, :]
    colr = lax.broadcasted_iota(jnp.int32, (R, CPAD), 1)
    cls = _first_argmax(L, colr)
    onehot = (colr == cls).astype(jnp.float32)

    counts = lax.dot_general(nmask, onehot, (((1,), (0,)), ((), ())),
                             precision=hi)
    total = jnp.sum(counts, axis=1, keepdims=True)
    frac = counts / total
    pur_ref[...] = (jnp.sum(-frac * jnp.log(frac + 1e-6), axis=1,
                            keepdims=True) / math.log(NUM_CLASSES))

    P = protos_ref[...]
    pn = lax.dot_general(ones_row, P * P, dn, precision=hi)
    disp = (-2.0 * lax.dot_general(Q, P, dn) + qn) + pn
    disp = jnp.maximum(disp, 1e-12)
    colq = lax.broadcasted_iota(jnp.int32, (BLK, CPAD), 1)
    disp = jnp.where(colq < NUM_CLASSES, disp, BIG)
    Lq = logits_ref[pl.ds(i * BLK, BLK), :]
    cls_q = _first_argmax(Lq, colq)
    dpred = jnp.sum(jnp.where(colq == cls_q, disp, 0.0), axis=1,
                    keepdims=True)
    rank = jnp.sum((disp < dpred).astype(jnp.float32), axis=1, keepdims=True)
    ent_ref[...] = rank / float(NUM_CLASSES - 1)


def _tc_main(feats_sel, logits_sel, protos_pad):
    return pl.pallas_call(
        _tc_body,
        grid=(GRID,),
        in_specs=[pl.BlockSpec((RPAD, D), lambda i: (0, 0)),
                  pl.BlockSpec((RPAD, CPAD), lambda i: (0, 0)),
                  pl.BlockSpec((CPAD, D), lambda i: (0, 0))],
        out_specs=[pl.BlockSpec((BLK, 1), lambda i: (i, 0)),
                   pl.BlockSpec((BLK, 1), lambda i: (i, 0))],
        out_shape=[jax.ShapeDtypeStruct((R, 1), jnp.float32),
                   jax.ShapeDtypeStruct((R, 1), jnp.float32)],
    )(feats_sel, logits_sel, protos_pad)


def _sc_scatter(pur, ent, idx):
    @functools.partial(
        pl.kernel,
        out_type=[jax.ShapeDtypeStruct((FULL,), jnp.float32),
                  jax.ShapeDtypeStruct((FULL,), jnp.float32)],
        mesh=plsc.VectorSubcoreMesh(**_MESH),
        scratch_types=[pltpu.VMEM((REGION,), jnp.float32),
                       pltpu.VMEM((R,), jnp.float32),
                       pltpu.VMEM((R,), jnp.int32)],
        compiler_params=pltpu.CompilerParams(needs_layout_passes=False),
    )
    def sk(pur_hbm, ent_hbm, idx_hbm, out_p, out_e, buf, vals, idxv):
        wid = lax.axis_index("s") * NC + lax.axis_index("c")

        def zero_chunk(i, carry):
            base = pl.multiple_of(i * 160, 160)
            for u in range(10):
                buf[pl.ds(base + u * 16, 16)] = jnp.zeros((16,), jnp.float32)
            return carry

        lax.fori_loop(0, REGION // 160, zero_chunk, 0)

        def scatter_into_buf(src_hbm):
            pltpu.sync_copy(idx_hbm, idxv)
            pltpu.sync_copy(src_hbm, vals)

            def body(j, carry):
                base = pl.multiple_of(j * 16, 16)
                plsc.store_scatter(buf, [idxv[pl.ds(base, 16)]],
                                   vals[pl.ds(base, 16)])
                return carry

            lax.fori_loop(0, R // 16, body, 0)

        own = pl.multiple_of(wid * REGION, REGION)

        @pl.when(wid >= 2)
        def _():
            pltpu.sync_copy(buf, out_p.at[pl.ds(own, REGION)])
            pltpu.sync_copy(buf, out_e.at[pl.ds(own, REGION)])

        @pl.when(wid == 0)
        def _():
            pltpu.sync_copy(buf, out_e.at[pl.ds(REGION, REGION)])
            scatter_into_buf(pur_hbm)
            pltpu.sync_copy(buf, out_p.at[pl.ds(0, REGION)])

        @pl.when(wid == 1)
        def _():
            pltpu.sync_copy(buf, out_p.at[pl.ds(REGION, REGION)])
            scatter_into_buf(ent_hbm)
            pltpu.sync_copy(buf, out_e.at[pl.ds(0, REGION)])

    return sk(pur, ent, idx)


def kernel(features_tensor, outputs, classes_prototypes):
    feats_hw = jnp.transpose(features_tensor[0], (1, 2, 0)).reshape(HW, D)
    logits_hw = jnp.transpose(outputs[0], (1, 2, 0)).reshape(HW, NUM_CLASSES)
    logits_hw = jnp.pad(logits_hw, ((0, 0), (0, CPAD - NUM_CLASSES)),
                        constant_values=NEG)
    protos = jnp.pad(classes_prototypes[0],
                     ((0, CPAD - NUM_CLASSES), (0, 0)))
    idx_pad = jnp.asarray(_SAMPLE_INDEX_PAD)

    feats_sel, logits_sel = _sc_gather(feats_hw, logits_hw, idx_pad)
    pur, ent = _tc_main(feats_sel, logits_sel, protos)

    idx = jnp.asarray(_SAMPLE_INDEX)
    full_p, full_e = _sc_scatter(pur.reshape(R), ent.reshape(R), idx)
    return (full_p.reshape(FULL_ROW, FULL_COL),
            full_e.reshape(FULL_ROW, FULL_COL))


# speedup vs baseline: 28.9404x; 1.1711x over previous
import functools
import math

import jax
import jax.numpy as jnp
import numpy as np
from jax import lax
from jax.experimental import pallas as pl
from jax.experimental.pallas import tpu as pltpu
from jax.experimental.pallas import tpu_sc as plsc

H, W = 135, 240
HW = H * W
R = 3600
D = 512
NUM_CLASSES = 19
NEIB = 9
CPAD = 128
CTC = 32
FULL_ROW, FULL_COL = 1080, 1920
FULL = FULL_ROW * FULL_COL
BLK = 400
GRID = R // BLK
NC, NS = 2, 16
NW = NC * NS
RPAD = 3840
B_PER_W = RPAD // NW
REGION = FULL // NW
NEG = -1e30
BIG = 3e38

def _sample_index_expr():
    return jnp.sort(jax.random.permutation(jax.random.key(42), HW)[:R])


def _compute_sample_index():
    try:
        cpu = jax.local_devices(backend="cpu")[0]
        with jax.default_device(cpu):
            return np.asarray(_sample_index_expr(), dtype=np.int32)
    except Exception:
        pass
    try:
        return np.asarray(_sample_index_expr(), dtype=np.int32)
    except Exception:
        return None


_SAMPLE_INDEX = _compute_sample_index()

_MESH = dict(core_axis_name="c", subcore_axis_name="s")


def _sc_gather(feats_hw, logits_hw, idx_pad):
    @functools.partial(
        pl.kernel,
        out_type=[jax.ShapeDtypeStruct((RPAD, D), jnp.float32),
                  jax.ShapeDtypeStruct((RPAD, CPAD), jnp.float32)],
        mesh=plsc.VectorSubcoreMesh(**_MESH),
        scratch_types=[pltpu.VMEM((B_PER_W,), jnp.int32),
                       pltpu.VMEM((B_PER_W, D), jnp.float32),
                       pltpu.VMEM((B_PER_W, CPAD), jnp.float32),
                       pltpu.SemaphoreType.DMA,
                       pltpu.SemaphoreType.DMA],
        compiler_params=pltpu.CompilerParams(needs_layout_passes=False),
    )
    def gk(feats_hbm, logits_hbm, idx_hbm, out_f, out_l, idx_v, rows_f,
           rows_l, sem_f, sem_l):
        wid = lax.axis_index("s") * NC + lax.axis_index("c")
        base = pl.multiple_of(wid * B_PER_W, B_PER_W)
        pltpu.sync_copy(idx_hbm.at[pl.ds(base, B_PER_W)], idx_v)
        cp_f = pltpu.async_copy(feats_hbm.at[idx_v], rows_f, sem_f)
        cp_l = pltpu.async_copy(logits_hbm.at[idx_v], rows_l, sem_l)
        cp_f.wait()
        cp_l.wait()
        pltpu.sync_copy(rows_f, out_f.at[pl.ds(base, B_PER_W)])
        pltpu.sync_copy(rows_l, out_l.at[pl.ds(base, B_PER_W)])

    return gk(feats_hw, logits_hw, idx_pad)


def _first_argmax(x, cols):
    rowmax = jnp.max(x, axis=1, keepdims=True)
    return jnp.min(jnp.where(x == rowmax, cols, CTC), axis=1, keepdims=True)


def _tc_body(feats_ref, logits_ref, protos_ref, pur_ref, ent_ref):
    i = pl.program_id(0)
    K = feats_ref[:R, :]
    Q = feats_ref[pl.ds(i * BLK, BLK), :]
    ones_row = jnp.ones((1, D), jnp.float32)
    dn = (((1,), (1,)), ((), ()))
    hi = lax.Precision.HIGHEST

    mm = lax.dot_general(Q, K, dn)
    qn = jnp.sum(Q * Q, axis=1, keepdims=True)
    kn = lax.dot_general(ones_row, K * K, dn, precision=hi)
    S = (-2.0 * mm + qn) + kn
    S = jnp.maximum(S, 1e-12)

    s_work = S
    m = jnp.min(s_work, axis=1, keepdims=True)
    for _ in range(NEIB - 1):
        s_work = jnp.where(s_work <= m, BIG, s_work)
        m = jnp.min(s_work, axis=1, keepdims=True)
    nmask = (S <= m).astype(jnp.bfloat16)

    L = logits_ref[:R, :]
    colr = lax.broadcasted_iota(jnp.int32, (R, CTC), 1)
    cls = _first_argmax(L, colr)
    onehot = (colr == cls).astype(jnp.bfloat16)

    counts = lax.dot_general(nmask, onehot, (((1,), (0,)), ((), ())),
                             preferred_element_type=jnp.float32)
    total = jnp.sum(counts, axis=1, keepdims=True)
    frac = counts / total
    pur_ref[...] = (jnp.sum(-frac * jnp.log(frac + 1e-6), axis=1,
                            keepdims=True) / math.log(NUM_CLASSES))

    P = protos_ref[...]
    pn = lax.dot_general(ones_row, P * P, dn, precision=hi)
    disp = (-2.0 * lax.dot_general(Q, P, dn) + qn) + pn
    disp = jnp.maximum(disp, 1e-12)
    colq = lax.broadcasted_iota(jnp.int32, (BLK, CTC), 1)
    disp = jnp.where(colq < NUM_CLASSES, disp, BIG)
    Lq = logits_ref[pl.ds(i * BLK, BLK), :]
    cls_q = _first_argmax(Lq, colq)
    dpred = jnp.sum(jnp.where(colq == cls_q, disp, 0.0), axis=1,
                    keepdims=True)
    rank = jnp.sum((disp < dpred).astype(jnp.float32), axis=1, keepdims=True)
    ent_ref[...] = rank / float(NUM_CLASSES - 1)


def _tc_main(feats_sel, logits_sel, protos_pad):
    return pl.pallas_call(
        _tc_body,
        grid=(GRID,),
        in_specs=[pl.BlockSpec((RPAD, D), lambda i: (0, 0)),
                  pl.BlockSpec((RPAD, CTC), lambda i: (0, 0)),
                  pl.BlockSpec((CTC, D), lambda i: (0, 0))],
        out_specs=[pl.BlockSpec((BLK, 1), lambda i: (i, 0)),
                   pl.BlockSpec((BLK, 1), lambda i: (i, 0))],
        out_shape=[jax.ShapeDtypeStruct((R, 1), jnp.float32),
                   jax.ShapeDtypeStruct((R, 1), jnp.float32)],
    )(feats_sel, logits_sel, protos_pad)


def _sc_scatter(pur, ent, idx):
    @functools.partial(
        pl.kernel,
        out_type=[jax.ShapeDtypeStruct((FULL,), jnp.float32),
                  jax.ShapeDtypeStruct((FULL,), jnp.float32)],
        mesh=plsc.VectorSubcoreMesh(**_MESH),
        scratch_types=[pltpu.VMEM((REGION,), jnp.float32),
                       pltpu.VMEM((R,), jnp.float32),
                       pltpu.VMEM((R,), jnp.int32)],
        compiler_params=pltpu.CompilerParams(needs_layout_passes=False),
    )
    def sk(pur_hbm, ent_hbm, idx_hbm, out_p, out_e, buf, vals, idxv):
        wid = lax.axis_index("s") * NC + lax.axis_index("c")

        def zero_chunk(i, carry):
            base = pl.multiple_of(i * 160, 160)
            for u in range(10):
                buf[pl.ds(base + u * 16, 16)] = jnp.zeros((16,), jnp.float32)
            return carry

        lax.fori_loop(0, REGION // 160, zero_chunk, 0)

        def scatter_into_buf(src_hbm):
            pltpu.sync_copy(idx_hbm, idxv)
            pltpu.sync_copy(src_hbm, vals)

            def body(j, carry):
                base = pl.multiple_of(j * 16, 16)
                plsc.store_scatter(buf, [idxv[pl.ds(base, 16)]],
                                   vals[pl.ds(base, 16)])
                return carry

            lax.fori_loop(0, R // 16, body, 0)

        own = pl.multiple_of(wid * REGION, REGION)

        @pl.when(wid >= 2)
        def _():
            pltpu.sync_copy(buf, out_p.at[pl.ds(own, REGION)])
            pltpu.sync_copy(buf, out_e.at[pl.ds(own, REGION)])

        @pl.when(wid == 0)
        def _():
            pltpu.sync_copy(buf, out_e.at[pl.ds(REGION, REGION)])
            scatter_into_buf(pur_hbm)
            pltpu.sync_copy(buf, out_p.at[pl.ds(0, REGION)])

        @pl.when(wid == 1)
        def _():
            pltpu.sync_copy(buf, out_p.at[pl.ds(REGION, REGION)])
            scatter_into_buf(ent_hbm)
            pltpu.sync_copy(buf, out_e.at[pl.ds(0, REGION)])

    return sk(pur, ent, idx)


def kernel(features_tensor, outputs, classes_prototypes):
    feats_hw = jnp.transpose(features_tensor[0], (1, 2, 0)).reshape(HW, D)
    logits_hw = jnp.transpose(outputs[0], (1, 2, 0)).reshape(HW, NUM_CLASSES)
    logits_hw = jnp.pad(logits_hw, ((0, 0), (0, CPAD - NUM_CLASSES)),
                        constant_values=NEG)
    protos = jnp.pad(classes_prototypes[0],
                     ((0, CTC - NUM_CLASSES), (0, 0)))
    if _SAMPLE_INDEX is not None:
        idx = jnp.asarray(_SAMPLE_INDEX)
    else:
        idx = _sample_index_expr().astype(jnp.int32)
    idx_pad = jnp.pad(idx, (0, RPAD - R))

    feats_sel, logits_sel = _sc_gather(feats_hw, logits_hw, idx_pad)
    pur, ent = _tc_main(feats_sel, logits_sel[:, :CTC], protos)

    full_p, full_e = _sc_scatter(pur.reshape(R), ent.reshape(R), idx)
    return (full_p.reshape(FULL_ROW, FULL_COL),
            full_e.reshape(FULL_ROW, FULL_COL))


# speedup vs baseline: 34.6838x; 1.1985x over previous
import functools
import math

import jax
import jax.numpy as jnp
import numpy as np
from jax import lax
from jax.experimental import pallas as pl
from jax.experimental.pallas import tpu as pltpu
from jax.experimental.pallas import tpu_sc as plsc

H, W = 135, 240
HW = H * W
R = 3600
D = 512
NUM_CLASSES = 19
NEIB = 9
CPAD = 128
CTC = 32
FULL_ROW, FULL_COL = 1080, 1920
FULL = FULL_ROW * FULL_COL
BLK = 600
GRID = R // BLK
NC, NS = 2, 16
NW = NC * NS
RPAD = 3840
B_PER_W = RPAD // NW
REGION = FULL // NW
NEG = -1e30
BIG = 3e38

def _sample_index_expr():
    return jnp.sort(jax.random.permutation(jax.random.key(42), HW)[:R])


def _compute_sample_index():
    try:
        cpu = jax.local_devices(backend="cpu")[0]
        with jax.default_device(cpu):
            return np.asarray(_sample_index_expr(), dtype=np.int32)
    except Exception:
        pass
    try:
        return np.asarray(_sample_index_expr(), dtype=np.int32)
    except Exception:
        return None


_SAMPLE_INDEX = _compute_sample_index()

_MESH = dict(core_axis_name="c", subcore_axis_name="s")


def _sc_gather(feats_hw, cls_hw, idx_pad):
    @functools.partial(
        pl.kernel,
        out_type=[jax.ShapeDtypeStruct((RPAD, D), jnp.float32),
                  jax.ShapeDtypeStruct((RPAD,), jnp.int32)],
        mesh=plsc.VectorSubcoreMesh(**_MESH),
        scratch_types=[pltpu.VMEM((128,), jnp.int32),
                       pltpu.VMEM((B_PER_W, D), jnp.float32),
                       pltpu.VMEM((HW,), jnp.int32),
                       pltpu.VMEM((128,), jnp.int32),
                       pltpu.SemaphoreType.DMA],
        compiler_params=pltpu.CompilerParams(needs_layout_passes=False),
    )
    def gk(feats_hbm, cls_hbm, idx_hbm, out_f, out_c, idx_v, rows_f,
           table_v, res_v, sem_f):
        wid = lax.axis_index("s") * NC + lax.axis_index("c")
        base = pl.multiple_of(wid * B_PER_W, B_PER_W)
        idx_v[pl.ds(112, 16)] = jnp.zeros((16,), jnp.int32)
        pltpu.sync_copy(idx_hbm.at[pl.ds(base, B_PER_W)], idx_v.at[pl.ds(0, B_PER_W)])
        cp_f = pltpu.async_copy(feats_hbm.at[idx_v.at[pl.ds(0, B_PER_W)]],
                                rows_f, sem_f)
        pltpu.sync_copy(cls_hbm, table_v)
        for j in range(8):
            res_v[pl.ds(j * 16, 16)] = plsc.load_gather(
                table_v, [idx_v[pl.ds(j * 16, 16)]])
        pltpu.sync_copy(res_v.at[pl.ds(0, B_PER_W)],
                        out_c.at[pl.ds(base, B_PER_W)])
        cp_f.wait()
        pltpu.sync_copy(rows_f, out_f.at[pl.ds(base, B_PER_W)])

    return gk(feats_hw, cls_hw, idx_pad)


def _cls_body(out_ref, cls_ref):
    Lg = out_ref[...]
    row = lax.broadcasted_iota(jnp.int32, (NUM_CLASSES, HW), 0)
    colmax = jnp.max(Lg, axis=0, keepdims=True)
    cls_ref[...] = jnp.min(jnp.where(Lg == colmax, row, NUM_CLASSES),
                           axis=0, keepdims=True)


def _tc_classids(outputs_hw):
    return pl.pallas_call(
        _cls_body,
        out_shape=jax.ShapeDtypeStruct((1, HW), jnp.int32),
    )(outputs_hw)


def _tc_body(feats_ref, cls_ref, protos_ref, pur_ref, ent_ref):
    i = pl.program_id(0)
    K = feats_ref[:R, :]
    Q = feats_ref[pl.ds(i * BLK, BLK), :]
    ones_row = jnp.ones((1, D), jnp.float32)
    dn = (((1,), (1,)), ((), ()))
    hi = lax.Precision.HIGHEST

    mm = lax.dot_general(Q, K, dn)
    qn = jnp.sum(Q * Q, axis=1, keepdims=True)
    kn = lax.dot_general(ones_row, K * K, dn, precision=hi)
    S = (-2.0 * mm + qn) + kn
    S = jnp.maximum(S, 1e-12)

    s_work = S
    m = jnp.min(s_work, axis=1, keepdims=True)
    for _ in range(NEIB - 1):
        s_work = jnp.where(s_work <= m, BIG, s_work)
        m = jnp.min(s_work, axis=1, keepdims=True)
    nmask = (S <= m).astype(jnp.bfloat16)

    colr = lax.broadcasted_iota(jnp.int32, (R, CTC), 1)
    onehot = (colr == cls_ref[:R, :]).astype(jnp.bfloat16)

    counts = lax.dot_general(nmask, onehot, (((1,), (0,)), ((), ())),
                             preferred_element_type=jnp.float32)
    total = jnp.sum(counts, axis=1, keepdims=True)
    frac = counts / total
    pur_ref[...] = (jnp.sum(-frac * jnp.log(frac + 1e-6), axis=1,
                            keepdims=True) / math.log(NUM_CLASSES))

    P = protos_ref[...]
    pn = lax.dot_general(ones_row, P * P, dn, precision=hi)
    disp = (-2.0 * lax.dot_general(Q, P, dn) + qn) + pn
    disp = jnp.maximum(disp, 1e-12)
    colq = lax.broadcasted_iota(jnp.int32, (BLK, CTC), 1)
    disp = jnp.where(colq < NUM_CLASSES, disp, BIG)
    cls_q = cls_ref[pl.ds(i * BLK, BLK), :]
    dpred = jnp.sum(jnp.where(colq == cls_q, disp, 0.0), axis=1,
                    keepdims=True)
    rank = jnp.sum((disp < dpred).astype(jnp.float32), axis=1, keepdims=True)
    ent_ref[...] = rank / float(NUM_CLASSES - 1)


def _tc_main(feats_sel, cls_sel, protos_pad):
    return pl.pallas_call(
        _tc_body,
        grid=(GRID,),
        in_specs=[pl.BlockSpec((RPAD, D), lambda i: (0, 0)),
                  pl.BlockSpec((RPAD, 1), lambda i: (0, 0)),
                  pl.BlockSpec((CTC, D), lambda i: (0, 0))],
        out_specs=[pl.BlockSpec((BLK, 1), lambda i: (i, 0)),
                   pl.BlockSpec((BLK, 1), lambda i: (i, 0))],
        out_shape=[jax.ShapeDtypeStruct((R, 1), jnp.float32),
                   jax.ShapeDtypeStruct((R, 1), jnp.float32)],
    )(feats_sel, cls_sel, protos_pad)


def _sc_scatter(pur, ent, idx):
    @functools.partial(
        pl.kernel,
        out_type=[jax.ShapeDtypeStruct((FULL,), jnp.float32),
                  jax.ShapeDtypeStruct((FULL,), jnp.float32)],
        mesh=plsc.VectorSubcoreMesh(**_MESH),
        scratch_types=[pltpu.VMEM((REGION,), jnp.float32),
                       pltpu.VMEM((R,), jnp.float32),
                       pltpu.VMEM((R,), jnp.int32)],
        compiler_params=pltpu.CompilerParams(needs_layout_passes=False),
    )
    def sk(pur_hbm, ent_hbm, idx_hbm, out_p, out_e, buf, vals, idxv):
        wid = lax.axis_index("s") * NC + lax.axis_index("c")

        def zero_chunk(i, carry):
            base = pl.multiple_of(i * 160, 160)
            for u in range(10):
                buf[pl.ds(base + u * 16, 16)] = jnp.zeros((16,), jnp.float32)
            return carry

        lax.fori_loop(0, REGION // 160, zero_chunk, 0)

        def scatter_into_buf(src_hbm):
            pltpu.sync_copy(idx_hbm, idxv)
            pltpu.sync_copy(src_hbm, vals)

            def body(j, carry):
                base = pl.multiple_of(j * 16, 16)
                plsc.store_scatter(buf, [idxv[pl.ds(base, 16)]],
                                   vals[pl.ds(base, 16)])
                return carry

            lax.fori_loop(0, R // 16, body, 0)

        own = pl.multiple_of(wid * REGION, REGION)

        @pl.when(wid >= 2)
        def _():
            pltpu.sync_copy(buf, out_p.at[pl.ds(own, REGION)])
            pltpu.sync_copy(buf, out_e.at[pl.ds(own, REGION)])

        @pl.when(wid == 0)
        def _():
            pltpu.sync_copy(buf, out_e.at[pl.ds(REGION, REGION)])
            scatter_into_buf(pur_hbm)
            pltpu.sync_copy(buf, out_p.at[pl.ds(0, REGION)])

        @pl.when(wid == 1)
        def _():
            pltpu.sync_copy(buf, out_p.at[pl.ds(REGION, REGION)])
            scatter_into_buf(ent_hbm)
            pltpu.sync_copy(buf, out_e.at[pl.ds(0, REGION)])

    return sk(pur, ent, idx)


def kernel(features_tensor, outputs, classes_prototypes):
    feats_hw = jnp.transpose(features_tensor[0], (1, 2, 0)).reshape(HW, D)
    cls_hw = _tc_classids(outputs[0].reshape(NUM_CLASSES, HW))
    protos = jnp.pad(classes_prototypes[0],
                     ((0, CTC - NUM_CLASSES), (0, 0)))
    if _SAMPLE_INDEX is not None:
        idx = jnp.asarray(_SAMPLE_INDEX)
    else:
        idx = _sample_index_expr().astype(jnp.int32)
    idx_pad = jnp.pad(idx, (0, RPAD - R))

    feats_sel, cls_sel = _sc_gather(feats_hw, cls_hw.reshape(HW), idx_pad)
    pur, ent = _tc_main(feats_sel, cls_sel.reshape(RPAD, 1), protos)

    full_p, full_e = _sc_scatter(pur.reshape(R), ent.reshape(R), idx)
    return (full_p.reshape(FULL_ROW, FULL_COL),
            full_e.reshape(FULL_ROW, FULL_COL))
